# Initial kernel scaffold; baseline (speedup 1.0000x reference)
import jax
import jax.numpy as jnp
from jax.experimental import pallas as pl


def kernel(hv_ftr, he_ftr, massive, p_ftr, q_ftr, edge_index, W_dis, b_dis, W_pe, b_pe, W_pv, b_pv):
    raise NotImplementedError("write your pallas kernel here")


# speedup vs baseline: 4.1253x; 4.1253x over previous
import functools

import jax
import jax.numpy as jnp
from jax import lax
from jax.experimental import pallas as pl
from jax.experimental.pallas import tpu as pltpu
from jax.experimental.pallas import tpu_sc as plsc

N = 10000
E = 160000
HV_DIM = 256
HE_DIM = 16
QD = 32
TAU = 0.25

NC = 2
NS = 16
NW = NC * NS
EPW = 5120
EP = NW * EPW
CH = 128
NCHUNK = EPW // CH
N_PAD = 10240
ROWS_PER_TILE = N_PAD // NS


def _softplus(x):
    return jnp.maximum(x, 0.0) + jnp.log1p(jnp.exp(-jnp.abs(x)))


def _node_body(hv, p, q, m, wpv, bpv, wdis, bdis, fv_o, lam_o, qn_o):
    h = hv[...]
    fv_o[...] = jnp.dot(h, wpv[...], preferred_element_type=jnp.float32) + bpv[...]
    t = jnp.dot(h, wdis[...], preferred_element_type=jnp.float32) + bdis[...]
    lam_o[...] = _softplus(t)
    qn_o[...] = q[...] + TAU * (p[...] / m[...])


def _node_call(hv, p, q, m, wpv, bpv, wdis, bdis):
    BN = 2000
    grid = N // BN
    return pl.pallas_call(
        _node_body,
        grid=(grid,),
        in_specs=[
            pl.BlockSpec((BN, HV_DIM), lambda i: (i, 0)),
            pl.BlockSpec((BN, QD), lambda i: (i, 0)),
            pl.BlockSpec((BN, QD), lambda i: (i, 0)),
            pl.BlockSpec((BN, 1), lambda i: (i, 0)),
            pl.BlockSpec((HV_DIM, QD), lambda i: (0, 0)),
            pl.BlockSpec((1, QD), lambda i: (0, 0)),
            pl.BlockSpec((HV_DIM, 1), lambda i: (0, 0)),
            pl.BlockSpec((1, 1), lambda i: (0, 0)),
        ],
        out_specs=[
            pl.BlockSpec((BN, QD), lambda i: (i, 0)),
            pl.BlockSpec((BN, 1), lambda i: (i, 0)),
            pl.BlockSpec((BN, QD), lambda i: (i, 0)),
        ],
        out_shape=[
            jax.ShapeDtypeStruct((N, QD), jnp.float32),
            jax.ShapeDtypeStruct((N, 1), jnp.float32),
            jax.ShapeDtypeStruct((N, QD), jnp.float32),
        ],
    )(hv, p, q, m, wpv, bpv.reshape(1, QD), wdis, bdis.reshape(1, 1))


def _edge_body(he, wpe, bpe, k_o):
    t = jnp.dot(he[...], wpe[...], preferred_element_type=jnp.float32) + bpe[...]
    k_o[...] = _softplus(t)


def _edge_call(he, wpe, bpe):
    BE = 8000
    grid = E // BE
    return pl.pallas_call(
        _edge_body,
        grid=(grid,),
        in_specs=[
            pl.BlockSpec((BE, HE_DIM), lambda i: (i, 0)),
            pl.BlockSpec((HE_DIM, 1), lambda i: (0, 0)),
            pl.BlockSpec((1, 1), lambda i: (0, 0)),
        ],
        out_specs=pl.BlockSpec((BE, 1), lambda i: (i, 0)),
        out_shape=jax.ShapeDtypeStruct((E, 1), jnp.float32),
    )(he, wpe, bpe.reshape(1, 1))


def _sc_body(q_hbm, srcp, dstp, kp, zeros_hbm, gout,
             is_v, id_v, k_v, qs_v, qd_v, mp_v, mm_v, gs, sem1, sem2):
    c = lax.axis_index("c")
    s = lax.axis_index("s")
    wid = s * NC + c

    pltpu.sync_copy(zeros_hbm.at[pl.ds(s * ROWS_PER_TILE, ROWS_PER_TILE)],
                    gs.at[pl.ds(s * ROWS_PER_TILE, ROWS_PER_TILE)])
    plsc.subcore_barrier()

    base0 = wid * EPW

    def chunk(i, carry):
        base = base0 + i * CH
        pltpu.sync_copy(srcp.at[pl.ds(base, CH)], is_v)
        pltpu.sync_copy(dstp.at[pl.ds(base, CH)], id_v)
        pltpu.sync_copy(kp.at[pl.ds(base, CH)], k_v)
        ga = pltpu.async_copy(q_hbm.at[is_v], qs_v, sem1)
        gb = pltpu.async_copy(q_hbm.at[id_v], qd_v, sem2)
        ga.wait()
        gb.wait()

        def group(g, carry2):
            kg = k_v[pl.ds(g * 16, 16)]
            for j in range(16):
                e = g * 16 + j
                kv = kg[j]
                for half in range(2):
                    sl = pl.ds(half * 16, 16)
                    d = qs_v[e, sl] - qd_v[e, sl]
                    msg = kv * d
                    mp_v[e, sl] = msg
                    mm_v[e, sl] = -msg
            return carry2

        lax.fori_loop(0, CH // 16, group, 0)

        pltpu.sync_copy(mp_v, gs.at[is_v], add=True)
        pltpu.sync_copy(mm_v, gs.at[id_v], add=True)
        return carry

    lax.fori_loop(0, NCHUNK, chunk, 0)
    plsc.subcore_barrier()

    pltpu.sync_copy(gs.at[pl.ds(s * ROWS_PER_TILE, ROWS_PER_TILE)],
                    gout.at[c, pl.ds(s * ROWS_PER_TILE, ROWS_PER_TILE)])


def _sc_call(q, srcp, dstp, kp, zeros):
    mesh = plsc.VectorSubcoreMesh(core_axis_name="c", subcore_axis_name="s")
    f = pl.kernel(
        _sc_body,
        out_type=jax.ShapeDtypeStruct((NC, N_PAD, QD), jnp.float32),
        mesh=mesh,
        scratch_types=[
            pltpu.VMEM((CH,), jnp.int32),
            pltpu.VMEM((CH,), jnp.int32),
            pltpu.VMEM((CH,), jnp.float32),
            pltpu.VMEM((CH, QD), jnp.float32),
            pltpu.VMEM((CH, QD), jnp.float32),
            pltpu.VMEM((CH, QD), jnp.float32),
            pltpu.VMEM((CH, QD), jnp.float32),
            pltpu.VMEM_SHARED((N_PAD, QD), jnp.float32),
            pltpu.SemaphoreType.DMA,
            pltpu.SemaphoreType.DMA,
        ],
        compiler_params=pltpu.CompilerParams(use_tc_tiling_on_sc=False),
    )
    return f(q, srcp, dstp, kp, zeros)


def _final_body(p, fv, lam, gp, pn_o):
    g = gp[0] + gp[1]
    dp = -(fv[...] + g) - lam[...] * p[...]
    pn_o[...] = p[...] + TAU * dp


def _final_call(p, fv, lam, gpart):
    BN = 2000
    grid = N // BN
    return pl.pallas_call(
        _final_body,
        grid=(grid,),
        in_specs=[
            pl.BlockSpec((BN, QD), lambda i: (i, 0)),
            pl.BlockSpec((BN, QD), lambda i: (i, 0)),
            pl.BlockSpec((BN, 1), lambda i: (i, 0)),
            pl.BlockSpec((NC, BN, QD), lambda i: (0, i, 0)),
        ],
        out_specs=pl.BlockSpec((BN, QD), lambda i: (i, 0)),
        out_shape=jax.ShapeDtypeStruct((N, QD), jnp.float32),
    )(p, fv, lam, gpart)


@jax.jit
def kernel(hv_ftr, he_ftr, massive, p_ftr, q_ftr, edge_index,
           W_dis, b_dis, W_pe, b_pe, W_pv, b_pv):
    fv, lam, q_new = _node_call(hv_ftr, p_ftr, q_ftr, massive,
                                W_pv, b_pv, W_dis, b_dis)
    k = _edge_call(he_ftr, W_pe, b_pe)

    pad = EP - E
    kp = jnp.pad(k[:, 0], (0, pad))
    srcp = jnp.pad(edge_index[0], (0, pad))
    dstp = jnp.pad(edge_index[1], (0, pad))
    zeros = jnp.zeros((N_PAD, QD), jnp.float32)

    gpart = _sc_call(q_ftr, srcp, dstp, kp, zeros)
    p_new = _final_call(p_ftr, fv, lam, gpart[:, :N])
    return (p_new, q_new)


# speedup vs baseline: 4.8762x; 1.1820x over previous
import functools

import jax
import jax.numpy as jnp
from jax import lax
from jax.experimental import pallas as pl
from jax.experimental.pallas import tpu as pltpu
from jax.experimental.pallas import tpu_sc as plsc

N = 10000
E = 160000
HV_DIM = 256
HE_DIM = 16
QD = 32
TAU = 0.25

NC = 2
NS = 16
NW = NC * NS
EPW = 5120
EP = NW * EPW
CH = 128
NCHUNK = EPW // CH
N_PAD = 10240
ROWS_PER_TILE = N_PAD // NS


def _softplus(x):
    return jnp.maximum(x, 0.0) + jnp.log1p(jnp.exp(-jnp.abs(x)))


def _node_body(hv, p, q, m, wpv, bpv, wdis, bdis, fv_o, lam_o, qn_o):
    h = hv[...]
    fv_o[...] = jnp.dot(h, wpv[...], preferred_element_type=jnp.float32) + bpv[...]
    t = jnp.dot(h, wdis[...], preferred_element_type=jnp.float32) + bdis[...]
    lam_o[...] = _softplus(t)
    qn_o[...] = q[...] + TAU * (p[...] / m[...])


def _node_call(hv, p, q, m, wpv, bpv, wdis, bdis):
    BN = 2000
    grid = N // BN
    return pl.pallas_call(
        _node_body,
        grid=(grid,),
        in_specs=[
            pl.BlockSpec((BN, HV_DIM), lambda i: (i, 0)),
            pl.BlockSpec((BN, QD), lambda i: (i, 0)),
            pl.BlockSpec((BN, QD), lambda i: (i, 0)),
            pl.BlockSpec((BN, 1), lambda i: (i, 0)),
            pl.BlockSpec((HV_DIM, QD), lambda i: (0, 0)),
            pl.BlockSpec((1, QD), lambda i: (0, 0)),
            pl.BlockSpec((HV_DIM, 1), lambda i: (0, 0)),
            pl.BlockSpec((1, 1), lambda i: (0, 0)),
        ],
        out_specs=[
            pl.BlockSpec((BN, QD), lambda i: (i, 0)),
            pl.BlockSpec((BN, 1), lambda i: (i, 0)),
            pl.BlockSpec((BN, QD), lambda i: (i, 0)),
        ],
        out_shape=[
            jax.ShapeDtypeStruct((N, QD), jnp.float32),
            jax.ShapeDtypeStruct((N, 1), jnp.float32),
            jax.ShapeDtypeStruct((N, QD), jnp.float32),
        ],
    )(hv, p, q, m, wpv, bpv.reshape(1, QD), wdis, bdis.reshape(1, 1))


def _edge_body(he, wpe, bpe, k_o):
    t = jnp.dot(he[...], wpe[...], preferred_element_type=jnp.float32) + bpe[...]
    k_o[...] = _softplus(t)


def _edge_call(he, wpe, bpe):
    BE = 8000
    grid = E // BE
    return pl.pallas_call(
        _edge_body,
        grid=(grid,),
        in_specs=[
            pl.BlockSpec((BE, HE_DIM), lambda i: (i, 0)),
            pl.BlockSpec((HE_DIM, 1), lambda i: (0, 0)),
            pl.BlockSpec((1, 1), lambda i: (0, 0)),
        ],
        out_specs=pl.BlockSpec((BE, 1), lambda i: (i, 0)),
        out_shape=jax.ShapeDtypeStruct((E, 1), jnp.float32),
    )(he, wpe, bpe.reshape(1, 1))


def _sc_body(q_hbm, srcp, dstp, kp, zeros_hbm, gout,
             is_all, id_all, k_all, qs_v, qd_v, mp_v, gs_p, gs_n,
             sem_s, sem_d):
    c = lax.axis_index("c")
    s = lax.axis_index("s")
    wid = s * NC + c

    pltpu.sync_copy(srcp.at[wid], is_all)
    pltpu.sync_copy(dstp.at[wid], id_all)
    pltpu.sync_copy(kp.at[wid], k_all)

    rsl = pl.ds(s * ROWS_PER_TILE, ROWS_PER_TILE)
    pltpu.sync_copy(zeros_hbm.at[rsl], gs_p.at[rsl])
    pltpu.sync_copy(zeros_hbm.at[rsl], gs_n.at[rsl])
    plsc.subcore_barrier()

    def fire(i):
        slot = lax.rem(i, 2)
        pltpu.async_copy(q_hbm.at[is_all.at[i]], qs_v.at[slot], sem_s)
        pltpu.async_copy(q_hbm.at[id_all.at[i]], qd_v.at[slot], sem_d)

    fire(0)

    def chunk(i, carry):
        slot = lax.rem(i, 2)

        @pl.when(i + 1 < NCHUNK)
        def _():
            fire(i + 1)

        pltpu.make_async_copy(q_hbm.at[is_all.at[i]], qs_v.at[slot], sem_s).wait()
        pltpu.make_async_copy(q_hbm.at[id_all.at[i]], qd_v.at[slot], sem_d).wait()

        def group(g, carry2):
            kg = k_all[i, pl.ds(g * 16, 16)]
            for j in range(16):
                e = g * 16 + j
                kv = kg[j]
                for half in range(2):
                    sl = pl.ds(half * 16, 16)
                    d = qs_v[slot, e, sl] - qd_v[slot, e, sl]
                    mp_v[e, sl] = kv * d
            return carry2

        lax.fori_loop(0, CH // 16, group, 0)

        pltpu.sync_copy(mp_v, gs_p.at[is_all.at[i]], add=True)
        pltpu.sync_copy(mp_v, gs_n.at[id_all.at[i]], add=True)
        return carry

    lax.fori_loop(0, NCHUNK, chunk, 0)
    plsc.subcore_barrier()

    pltpu.sync_copy(gs_p.at[rsl], gout.at[c, 0, rsl])
    pltpu.sync_copy(gs_n.at[rsl], gout.at[c, 1, rsl])


def _sc_call(q, srcp, dstp, kp, zeros):
    mesh = plsc.VectorSubcoreMesh(core_axis_name="c", subcore_axis_name="s")
    f = pl.kernel(
        _sc_body,
        out_type=jax.ShapeDtypeStruct((NC, 2, N_PAD, QD), jnp.float32),
        mesh=mesh,
        scratch_types=[
            pltpu.VMEM((NCHUNK, CH), jnp.int32),
            pltpu.VMEM((NCHUNK, CH), jnp.int32),
            pltpu.VMEM((NCHUNK, CH), jnp.float32),
            pltpu.VMEM((2, CH, QD), jnp.float32),
            pltpu.VMEM((2, CH, QD), jnp.float32),
            pltpu.VMEM((CH, QD), jnp.float32),
            pltpu.VMEM_SHARED((N_PAD, QD), jnp.float32),
            pltpu.VMEM_SHARED((N_PAD, QD), jnp.float32),
            pltpu.SemaphoreType.DMA,
            pltpu.SemaphoreType.DMA,
        ],
        compiler_params=pltpu.CompilerParams(use_tc_tiling_on_sc=False),
    )
    return f(q, srcp, dstp, kp, zeros)


def _final_body(p, fv, lam, gp, pn_o):
    g = gp[0, 0] + gp[1, 0] - gp[0, 1] - gp[1, 1]
    dp = -(fv[...] + g) - lam[...] * p[...]
    pn_o[...] = p[...] + TAU * dp


def _final_call(p, fv, lam, gpart):
    BN = 2000
    grid = N // BN
    return pl.pallas_call(
        _final_body,
        grid=(grid,),
        in_specs=[
            pl.BlockSpec((BN, QD), lambda i: (i, 0)),
            pl.BlockSpec((BN, QD), lambda i: (i, 0)),
            pl.BlockSpec((BN, 1), lambda i: (i, 0)),
            pl.BlockSpec((NC, 2, BN, QD), lambda i: (0, 0, i, 0)),
        ],
        out_specs=pl.BlockSpec((BN, QD), lambda i: (i, 0)),
        out_shape=jax.ShapeDtypeStruct((N, QD), jnp.float32),
    )(p, fv, lam, gpart)


@jax.jit
def kernel(hv_ftr, he_ftr, massive, p_ftr, q_ftr, edge_index,
           W_dis, b_dis, W_pe, b_pe, W_pv, b_pv):
    fv, lam, q_new = _node_call(hv_ftr, p_ftr, q_ftr, massive,
                                W_pv, b_pv, W_dis, b_dis)
    k = _edge_call(he_ftr, W_pe, b_pe)

    pad = EP - E
    kp = jnp.pad(k[:, 0], (0, pad)).reshape(NW, NCHUNK, CH)
    srcp = jnp.pad(edge_index[0], (0, pad)).reshape(NW, NCHUNK, CH)
    dstp = jnp.pad(edge_index[1], (0, pad)).reshape(NW, NCHUNK, CH)
    zeros = jnp.zeros((N_PAD, QD), jnp.float32)

    gpart = _sc_call(q_ftr, srcp, dstp, kp, zeros)
    p_new = _final_call(p_ftr, fv, lam, gpart[:, :, :N])
    return (p_new, q_new)


# speedup vs baseline: 6.1917x; 1.2698x over previous
import functools

import jax
import jax.numpy as jnp
from jax import lax
from jax.experimental import pallas as pl
from jax.experimental.pallas import tpu as pltpu
from jax.experimental.pallas import tpu_sc as plsc

N = 10000
E = 160000
HV_DIM = 256
HE_DIM = 16
QD = 32
TAU = 0.25

NC = 2
NS = 16
NW = NC * NS
EPW = 5120
EP = NW * EPW
CH = 128
NCHUNK = EPW // CH
N_PAD = 10240
ROWS_PER_TILE = N_PAD // NS


def _softplus(x):
    return jnp.maximum(x, 0.0) + jnp.log1p(jnp.exp(-jnp.abs(x)))


def _node_body(hv, p, q, m, wpv, bpv, wdis, bdis, fv_o, lam_o, qn_o):
    h = hv[...]
    fv_o[...] = jnp.dot(h, wpv[...], preferred_element_type=jnp.float32) + bpv[...]
    t = jnp.dot(h, wdis[...], preferred_element_type=jnp.float32) + bdis[...]
    lam_o[...] = _softplus(t)
    qn_o[...] = q[...] + TAU * (p[...] / m[...])


def _node_call(hv, p, q, m, wpv, bpv, wdis, bdis):
    BN = 2000
    grid = N // BN
    return pl.pallas_call(
        _node_body,
        grid=(grid,),
        in_specs=[
            pl.BlockSpec((BN, HV_DIM), lambda i: (i, 0)),
            pl.BlockSpec((BN, QD), lambda i: (i, 0)),
            pl.BlockSpec((BN, QD), lambda i: (i, 0)),
            pl.BlockSpec((BN, 1), lambda i: (i, 0)),
            pl.BlockSpec((HV_DIM, QD), lambda i: (0, 0)),
            pl.BlockSpec((1, QD), lambda i: (0, 0)),
            pl.BlockSpec((HV_DIM, 1), lambda i: (0, 0)),
            pl.BlockSpec((1, 1), lambda i: (0, 0)),
        ],
        out_specs=[
            pl.BlockSpec((BN, QD), lambda i: (i, 0)),
            pl.BlockSpec((BN, 1), lambda i: (i, 0)),
            pl.BlockSpec((BN, QD), lambda i: (i, 0)),
        ],
        out_shape=[
            jax.ShapeDtypeStruct((N, QD), jnp.float32),
            jax.ShapeDtypeStruct((N, 1), jnp.float32),
            jax.ShapeDtypeStruct((N, QD), jnp.float32),
        ],
    )(hv, p, q, m, wpv, bpv.reshape(1, QD), wdis, bdis.reshape(1, 1))


E_ROWS = E // CH
EP_ROWS = EP // CH


def _edge_body(he2, wmat, bpe, k_o):
    t = jnp.dot(he2[...], wmat[...], preferred_element_type=jnp.float32) + bpe[...]
    k = _softplus(t)
    k_o[...] = jnp.concatenate(
        [k, jnp.zeros((EP_ROWS - E_ROWS, CH), jnp.float32)], axis=0)


def _edge_call(he, wpe, bpe):
    he2 = he.reshape(E_ROWS, CH * HE_DIM)
    wmat = jnp.kron(jnp.eye(CH, dtype=jnp.float32), wpe)
    return pl.pallas_call(
        _edge_body,
        grid=(1,),
        in_specs=[
            pl.BlockSpec((E_ROWS, CH * HE_DIM), lambda i: (0, 0)),
            pl.BlockSpec((CH * HE_DIM, CH), lambda i: (0, 0)),
            pl.BlockSpec((1, 1), lambda i: (0, 0)),
        ],
        out_specs=pl.BlockSpec((EP_ROWS, CH), lambda i: (0, 0)),
        out_shape=jax.ShapeDtypeStruct((EP_ROWS, CH), jnp.float32),
    )(he2, wmat, bpe.reshape(1, 1))


def _sc_body(q_hbm, srcp, dstp, kp, zeros_hbm, gout,
             is_all, id_all, k_all, qs_v, qd_v, mp_v, gs_p, gs_n,
             sem_s, sem_d):
    c = lax.axis_index("c")
    s = lax.axis_index("s")
    wid = s * NC + c

    row0 = wid * NCHUNK
    pltpu.sync_copy(srcp.at[pl.ds(row0, NCHUNK)], is_all)
    pltpu.sync_copy(dstp.at[pl.ds(row0, NCHUNK)], id_all)
    pltpu.sync_copy(kp.at[pl.ds(row0, NCHUNK)], k_all)

    rsl = pl.ds(s * ROWS_PER_TILE, ROWS_PER_TILE)
    pltpu.sync_copy(zeros_hbm.at[rsl], gs_p.at[rsl])
    pltpu.sync_copy(zeros_hbm.at[rsl], gs_n.at[rsl])
    plsc.subcore_barrier()

    def fire(i):
        slot = lax.rem(i, 2)
        pltpu.async_copy(q_hbm.at[is_all.at[i]], qs_v.at[slot], sem_s)
        pltpu.async_copy(q_hbm.at[id_all.at[i]], qd_v.at[slot], sem_d)

    fire(0)

    def chunk(i, carry):
        slot = lax.rem(i, 2)

        @pl.when(i + 1 < NCHUNK)
        def _():
            fire(i + 1)

        pltpu.make_async_copy(q_hbm.at[is_all.at[i]], qs_v.at[slot], sem_s).wait()
        pltpu.make_async_copy(q_hbm.at[id_all.at[i]], qd_v.at[slot], sem_d).wait()

        def group(g, carry2):
            kg = k_all[i, pl.ds(g * 16, 16)]
            for j in range(16):
                e = g * 16 + j
                kv = kg[j]
                for half in range(2):
                    sl = pl.ds(half * 16, 16)
                    d = qs_v[slot, e, sl] - qd_v[slot, e, sl]
                    mp_v[e, sl] = kv * d
            return carry2

        lax.fori_loop(0, CH // 16, group, 0)

        pltpu.sync_copy(mp_v, gs_p.at[is_all.at[i]], add=True)
        pltpu.sync_copy(mp_v, gs_n.at[id_all.at[i]], add=True)
        return carry

    lax.fori_loop(0, NCHUNK, chunk, 0)
    plsc.subcore_barrier()

    pltpu.sync_copy(gs_p.at[rsl], gout.at[c, 0, rsl])
    pltpu.sync_copy(gs_n.at[rsl], gout.at[c, 1, rsl])


def _sc_call(q, srcp, dstp, kp, zeros):
    mesh = plsc.VectorSubcoreMesh(core_axis_name="c", subcore_axis_name="s")
    f = pl.kernel(
        _sc_body,
        out_type=jax.ShapeDtypeStruct((NC, 2, N_PAD, QD), jnp.float32),
        mesh=mesh,
        scratch_types=[
            pltpu.VMEM((NCHUNK, CH), jnp.int32),
            pltpu.VMEM((NCHUNK, CH), jnp.int32),
            pltpu.VMEM((NCHUNK, CH), jnp.float32),
            pltpu.VMEM((2, CH, QD), jnp.float32),
            pltpu.VMEM((2, CH, QD), jnp.float32),
            pltpu.VMEM((CH, QD), jnp.float32),
            pltpu.VMEM_SHARED((N_PAD, QD), jnp.float32),
            pltpu.VMEM_SHARED((N_PAD, QD), jnp.float32),
            pltpu.SemaphoreType.DMA,
            pltpu.SemaphoreType.DMA,
        ],
        compiler_params=pltpu.CompilerParams(use_tc_tiling_on_sc=False),
    )
    return f(q, srcp, dstp, kp, zeros)


def _final_body(p, fv, lam, gp, pn_o):
    g = gp[0, 0] + gp[1, 0] - gp[0, 1] - gp[1, 1]
    dp = -(fv[...] + g) - lam[...] * p[...]
    pn_o[...] = p[...] + TAU * dp


def _final_call(p, fv, lam, gpart):
    BN = 2000
    grid = N // BN
    return pl.pallas_call(
        _final_body,
        grid=(grid,),
        in_specs=[
            pl.BlockSpec((BN, QD), lambda i: (i, 0)),
            pl.BlockSpec((BN, QD), lambda i: (i, 0)),
            pl.BlockSpec((BN, 1), lambda i: (i, 0)),
            pl.BlockSpec((NC, 2, BN, QD), lambda i: (0, 0, i, 0)),
        ],
        out_specs=pl.BlockSpec((BN, QD), lambda i: (i, 0)),
        out_shape=jax.ShapeDtypeStruct((N, QD), jnp.float32),
    )(p, fv, lam, gpart)


@jax.jit
def kernel(hv_ftr, he_ftr, massive, p_ftr, q_ftr, edge_index,
           W_dis, b_dis, W_pe, b_pe, W_pv, b_pv):
    fv, lam, q_new = _node_call(hv_ftr, p_ftr, q_ftr, massive,
                                W_pv, b_pv, W_dis, b_dis)
    k = _edge_call(he_ftr, W_pe, b_pe)

    pad = EP - E
    srcp = jnp.pad(edge_index[0], (0, pad)).reshape(EP_ROWS, CH)
    dstp = jnp.pad(edge_index[1], (0, pad)).reshape(EP_ROWS, CH)
    zeros = jnp.zeros((N_PAD, QD), jnp.float32)

    gpart = _sc_call(q_ftr, srcp, dstp, k, zeros)
    p_new = _final_call(p_ftr, fv, lam, gpart)
    return (p_new, q_new)


# speedup vs baseline: 6.5667x; 1.0606x over previous
import functools

import jax
import jax.numpy as jnp
from jax import lax
from jax.experimental import pallas as pl
from jax.experimental.pallas import tpu as pltpu
from jax.experimental.pallas import tpu_sc as plsc

N = 10000
E = 160000
HV_DIM = 256
HE_DIM = 16
QD = 32
TAU = 0.25

NC = 2
NS = 16
NW = NC * NS
EPW = 5120
EP = NW * EPW
CH = 128
NCHUNK = EPW // CH
N_PAD = 10240
ROWS_PER_TILE = N_PAD // NS


def _softplus(x):
    return jnp.maximum(x, 0.0) + jnp.log1p(jnp.exp(-jnp.abs(x)))


def _node_body(hv, p, q, m, wpv, bpv, wdis, bdis, fv_o, lam_o, qn_o):
    h = hv[...]
    fv_o[...] = jnp.dot(h, wpv[...], preferred_element_type=jnp.float32) + bpv[...]
    t = jnp.dot(h, wdis[...], preferred_element_type=jnp.float32) + bdis[...]
    lam_o[...] = _softplus(t)
    qn_o[...] = q[...] + TAU * (p[...] / m[...])


def _node_call(hv, p, q, m, wpv, bpv, wdis, bdis):
    BN = 2000
    grid = N // BN
    return pl.pallas_call(
        _node_body,
        grid=(grid,),
        in_specs=[
            pl.BlockSpec((BN, HV_DIM), lambda i: (i, 0)),
            pl.BlockSpec((BN, QD), lambda i: (i, 0)),
            pl.BlockSpec((BN, QD), lambda i: (i, 0)),
            pl.BlockSpec((BN, 1), lambda i: (i, 0)),
            pl.BlockSpec((HV_DIM, QD), lambda i: (0, 0)),
            pl.BlockSpec((1, QD), lambda i: (0, 0)),
            pl.BlockSpec((HV_DIM, 1), lambda i: (0, 0)),
            pl.BlockSpec((1, 1), lambda i: (0, 0)),
        ],
        out_specs=[
            pl.BlockSpec((BN, QD), lambda i: (i, 0)),
            pl.BlockSpec((BN, 1), lambda i: (i, 0)),
            pl.BlockSpec((BN, QD), lambda i: (i, 0)),
        ],
        out_shape=[
            jax.ShapeDtypeStruct((N, QD), jnp.float32),
            jax.ShapeDtypeStruct((N, 1), jnp.float32),
            jax.ShapeDtypeStruct((N, QD), jnp.float32),
        ],
    )(hv, p, q, m, wpv, bpv.reshape(1, QD), wdis, bdis.reshape(1, 1))


E_ROWS = E // CH
EP_ROWS = EP // CH


def _edge_body(he1, wmat, bpe, k_o):
    he2 = he1[...].reshape(E_ROWS, CH * HE_DIM)
    t = jnp.dot(he2, wmat[...], preferred_element_type=jnp.float32) + bpe[...]
    k = _softplus(t)
    k_o[...] = jnp.concatenate(
        [k, jnp.zeros((EP_ROWS - E_ROWS, CH), jnp.float32)], axis=0)


def _edge_call(he, wpe, bpe):
    he1 = he.reshape(E * HE_DIM)
    wmat = jnp.kron(jnp.eye(CH, dtype=jnp.float32), wpe)
    return pl.pallas_call(
        _edge_body,
        grid=(1,),
        in_specs=[
            pl.BlockSpec((E * HE_DIM,), lambda i: (0,)),
            pl.BlockSpec((CH * HE_DIM, CH), lambda i: (0, 0)),
            pl.BlockSpec((1, 1), lambda i: (0, 0)),
        ],
        out_specs=pl.BlockSpec((EP_ROWS, CH), lambda i: (0, 0)),
        out_shape=jax.ShapeDtypeStruct((EP_ROWS, CH), jnp.float32),
    )(he1, wmat, bpe.reshape(1, 1))


def _sc_body(q_hbm, srcp, dstp, kp, zeros_hbm, gout,
             is_all, id_all, k_all, qs_v, qd_v, mp_v, gs_p, gs_n,
             sem_s, sem_d):
    c = lax.axis_index("c")
    s = lax.axis_index("s")
    wid = s * NC + c

    row0 = wid * NCHUNK
    pltpu.sync_copy(srcp.at[pl.ds(row0, NCHUNK)], is_all)
    pltpu.sync_copy(dstp.at[pl.ds(row0, NCHUNK)], id_all)
    pltpu.sync_copy(kp.at[pl.ds(row0, NCHUNK)], k_all)

    rsl = pl.ds(s * ROWS_PER_TILE, ROWS_PER_TILE)
    pltpu.sync_copy(zeros_hbm.at[rsl], gs_p.at[rsl])
    pltpu.sync_copy(zeros_hbm.at[rsl], gs_n.at[rsl])
    plsc.subcore_barrier()

    def fire(i):
        slot = lax.rem(i, 2)
        pltpu.async_copy(q_hbm.at[is_all.at[i]], qs_v.at[slot], sem_s)
        pltpu.async_copy(q_hbm.at[id_all.at[i]], qd_v.at[slot], sem_d)

    fire(0)

    def chunk(i, carry):
        slot = lax.rem(i, 2)

        @pl.when(i + 1 < NCHUNK)
        def _():
            fire(i + 1)

        pltpu.make_async_copy(q_hbm.at[is_all.at[i]], qs_v.at[slot], sem_s).wait()
        pltpu.make_async_copy(q_hbm.at[id_all.at[i]], qd_v.at[slot], sem_d).wait()

        def group(g, carry2):
            kg = k_all[i, pl.ds(g * 16, 16)]
            for j in range(16):
                e = g * 16 + j
                kv = kg[j]
                for half in range(2):
                    sl = pl.ds(half * 16, 16)
                    d = qs_v[slot, e, sl] - qd_v[slot, e, sl]
                    mp_v[e, sl] = kv * d
            return carry2

        lax.fori_loop(0, CH // 16, group, 0)

        pltpu.sync_copy(mp_v, gs_p.at[is_all.at[i]], add=True)
        pltpu.sync_copy(mp_v, gs_n.at[id_all.at[i]], add=True)
        return carry

    lax.fori_loop(0, NCHUNK, chunk, 0)
    plsc.subcore_barrier()

    pltpu.sync_copy(gs_p.at[rsl], gout.at[c, 0, rsl])
    pltpu.sync_copy(gs_n.at[rsl], gout.at[c, 1, rsl])


def _sc_call(q, srcp, dstp, kp, zeros):
    mesh = plsc.VectorSubcoreMesh(core_axis_name="c", subcore_axis_name="s")
    f = pl.kernel(
        _sc_body,
        out_type=jax.ShapeDtypeStruct((NC, 2, N_PAD, QD), jnp.float32),
        mesh=mesh,
        scratch_types=[
            pltpu.VMEM((NCHUNK, CH), jnp.int32),
            pltpu.VMEM((NCHUNK, CH), jnp.int32),
            pltpu.VMEM((NCHUNK, CH), jnp.float32),
            pltpu.VMEM((2, CH, QD), jnp.float32),
            pltpu.VMEM((2, CH, QD), jnp.float32),
            pltpu.VMEM((CH, QD), jnp.float32),
            pltpu.VMEM_SHARED((N_PAD, QD), jnp.float32),
            pltpu.VMEM_SHARED((N_PAD, QD), jnp.float32),
            pltpu.SemaphoreType.DMA,
            pltpu.SemaphoreType.DMA,
        ],
        compiler_params=pltpu.CompilerParams(use_tc_tiling_on_sc=False),
    )
    return f(q, srcp, dstp, kp, zeros)


def _final_body(p, fv, lam, gp, pn_o):
    g = gp[0, 0] + gp[1, 0] - gp[0, 1] - gp[1, 1]
    dp = -(fv[...] + g) - lam[...] * p[...]
    pn_o[...] = p[...] + TAU * dp


def _final_call(p, fv, lam, gpart):
    BN = 2000
    grid = N // BN
    return pl.pallas_call(
        _final_body,
        grid=(grid,),
        in_specs=[
            pl.BlockSpec((BN, QD), lambda i: (i, 0)),
            pl.BlockSpec((BN, QD), lambda i: (i, 0)),
            pl.BlockSpec((BN, 1), lambda i: (i, 0)),
            pl.BlockSpec((NC, 2, BN, QD), lambda i: (0, 0, i, 0)),
        ],
        out_specs=pl.BlockSpec((BN, QD), lambda i: (i, 0)),
        out_shape=jax.ShapeDtypeStruct((N, QD), jnp.float32),
    )(p, fv, lam, gpart)


@jax.jit
def kernel(hv_ftr, he_ftr, massive, p_ftr, q_ftr, edge_index,
           W_dis, b_dis, W_pe, b_pe, W_pv, b_pv):
    fv, lam, q_new = _node_call(hv_ftr, p_ftr, q_ftr, massive,
                                W_pv, b_pv, W_dis, b_dis)
    k = _edge_call(he_ftr, W_pe, b_pe)

    pad = EP - E
    srcp = jnp.pad(edge_index[0], (0, pad)).reshape(EP_ROWS, CH)
    dstp = jnp.pad(edge_index[1], (0, pad)).reshape(EP_ROWS, CH)
    zeros = jnp.zeros((N_PAD, QD), jnp.float32)

    gpart = _sc_call(q_ftr, srcp, dstp, k, zeros)
    p_new = _final_call(p_ftr, fv, lam, gpart)
    return (p_new, q_new)


# speedup vs baseline: 8.1716x; 1.2444x over previous
import functools

import jax
import jax.numpy as jnp
from jax import lax
from jax.experimental import pallas as pl
from jax.experimental.pallas import tpu as pltpu
from jax.experimental.pallas import tpu_sc as plsc

N = 10000
E = 160000
HV_DIM = 256
HE_DIM = 16
QD = 32
TAU = 0.25

NC = 2
NS = 16
NW = NC * NS
EPW = 5120
EP = NW * EPW
CH = 128
NCHUNK = EPW // CH
N_PAD = 10240
ROWS_PER_TILE = N_PAD // NS


def _softplus(x):
    return jnp.maximum(x, 0.0) + jnp.log1p(jnp.exp(-jnp.abs(x)))


def _node_body(hv, p, q, m, wpv, bpv, wdis, bdis, fv_o, lam_o, qn_o):
    h = hv[...]
    fv_o[...] = jnp.dot(h, wpv[...], preferred_element_type=jnp.float32) + bpv[...]
    t = jnp.dot(h, wdis[...], preferred_element_type=jnp.float32) + bdis[...]
    lam_o[...] = _softplus(t)
    qn_o[...] = q[...] + TAU * (p[...] / m[...])


def _node_call(hv, p, q, m, wpv, bpv, wdis, bdis):
    BN = 2000
    grid = N // BN
    return pl.pallas_call(
        _node_body,
        grid=(grid,),
        in_specs=[
            pl.BlockSpec((BN, HV_DIM), lambda i: (i, 0)),
            pl.BlockSpec((BN, QD), lambda i: (i, 0)),
            pl.BlockSpec((BN, QD), lambda i: (i, 0)),
            pl.BlockSpec((BN, 1), lambda i: (i, 0)),
            pl.BlockSpec((HV_DIM, QD), lambda i: (0, 0)),
            pl.BlockSpec((1, QD), lambda i: (0, 0)),
            pl.BlockSpec((HV_DIM, 1), lambda i: (0, 0)),
            pl.BlockSpec((1, 1), lambda i: (0, 0)),
        ],
        out_specs=[
            pl.BlockSpec((BN, QD), lambda i: (i, 0)),
            pl.BlockSpec((BN, 1), lambda i: (i, 0)),
            pl.BlockSpec((BN, QD), lambda i: (i, 0)),
        ],
        out_shape=[
            jax.ShapeDtypeStruct((N, QD), jnp.float32),
            jax.ShapeDtypeStruct((N, 1), jnp.float32),
            jax.ShapeDtypeStruct((N, QD), jnp.float32),
        ],
    )(hv, p, q, m, wpv, bpv.reshape(1, QD), wdis, bdis.reshape(1, 1))


E_ROWS = E // CH
EP_ROWS = EP // CH


def _edge_body(het, w, b, k_o):
    x = het[...]
    t = jnp.sum(x * w[...], axis=0) + b[...][0]
    k = _softplus(t)
    k_o[...] = jnp.concatenate([k, jnp.zeros((EP - E,), jnp.float32)])


def _edge_call(he, wpe, bpe):
    het = he.T
    return pl.pallas_call(
        _edge_body,
        grid=(1,),
        in_specs=[
            pl.BlockSpec((HE_DIM, E), lambda i: (0, 0)),
            pl.BlockSpec((HE_DIM, 1), lambda i: (0, 0)),
            pl.BlockSpec((1,), lambda i: (0,)),
        ],
        out_specs=pl.BlockSpec((EP,), lambda i: (0,)),
        out_shape=jax.ShapeDtypeStruct((EP,), jnp.float32),
    )(het, wpe, bpe)


def _sc_body(q_hbm, srcp, dstp, kp, zeros_hbm, gout,
             is_all, id_all, k_all, qs_v, qd_v, mp_v, gs_p, gs_n,
             sem_s, sem_d, sem_u):
    c = lax.axis_index("c")
    s = lax.axis_index("s")
    wid = s * NC + c

    row0 = wid * NCHUNK
    pltpu.sync_copy(srcp.at[pl.ds(row0, NCHUNK)], is_all)
    pltpu.sync_copy(dstp.at[pl.ds(row0, NCHUNK)], id_all)
    pltpu.sync_copy(kp.at[pl.ds(wid * EPW, EPW)], k_all)

    rsl = pl.ds(s * ROWS_PER_TILE, ROWS_PER_TILE)
    pltpu.sync_copy(zeros_hbm.at[rsl], gs_p.at[rsl])
    pltpu.sync_copy(zeros_hbm.at[rsl], gs_n.at[rsl])
    plsc.subcore_barrier()

    def fire(i):
        slot = lax.rem(i, 2)
        pltpu.async_copy(q_hbm.at[is_all.at[i]], qs_v.at[slot], sem_s)
        pltpu.async_copy(q_hbm.at[id_all.at[i]], qd_v.at[slot], sem_d)

    fire(0)

    def chunk(i, carry):
        slot = lax.rem(i, 2)

        @pl.when(i + 1 < NCHUNK)
        def _():
            fire(i + 1)

        pltpu.make_async_copy(q_hbm.at[is_all.at[i]], qs_v.at[slot], sem_s).wait()
        pltpu.make_async_copy(q_hbm.at[id_all.at[i]], qd_v.at[slot], sem_d).wait()

        @pl.when(i >= 2)
        def _():
            pltpu.make_async_copy(mp_v.at[slot], gs_p.at[is_all.at[i]], sem_u).wait()
            pltpu.make_async_copy(mp_v.at[slot], gs_n.at[id_all.at[i]], sem_u).wait()

        def group(g, carry2):
            kg = k_all[pl.ds(i * CH + g * 16, 16)]
            for j in range(16):
                e = g * 16 + j
                kv = kg[j]
                for half in range(2):
                    sl = pl.ds(half * 16, 16)
                    d = qs_v[slot, e, sl] - qd_v[slot, e, sl]
                    mp_v[slot, e, sl] = kv * d
            return carry2

        lax.fori_loop(0, CH // 16, group, 0)

        pltpu.async_copy(mp_v.at[slot], gs_p.at[is_all.at[i]], sem_u, add=True)
        pltpu.async_copy(mp_v.at[slot], gs_n.at[id_all.at[i]], sem_u, add=True)
        return carry

    lax.fori_loop(0, NCHUNK, chunk, 0)

    for _ in range(4):
        pltpu.make_async_copy(mp_v.at[0], gs_p.at[is_all.at[0]], sem_u).wait()
    plsc.subcore_barrier()

    pltpu.sync_copy(gs_p.at[rsl], gout.at[c, 0, rsl])
    pltpu.sync_copy(gs_n.at[rsl], gout.at[c, 1, rsl])


def _sc_call(q, srcp, dstp, kp, zeros):
    mesh = plsc.VectorSubcoreMesh(core_axis_name="c", subcore_axis_name="s")
    f = pl.kernel(
        _sc_body,
        out_type=jax.ShapeDtypeStruct((NC, 2, N_PAD, QD), jnp.float32),
        mesh=mesh,
        scratch_types=[
            pltpu.VMEM((NCHUNK, CH), jnp.int32),
            pltpu.VMEM((NCHUNK, CH), jnp.int32),
            pltpu.VMEM((EPW,), jnp.float32),
            pltpu.VMEM((2, CH, QD), jnp.float32),
            pltpu.VMEM((2, CH, QD), jnp.float32),
            pltpu.VMEM((2, CH, QD), jnp.float32),
            pltpu.VMEM_SHARED((N_PAD, QD), jnp.float32),
            pltpu.VMEM_SHARED((N_PAD, QD), jnp.float32),
            pltpu.SemaphoreType.DMA,
            pltpu.SemaphoreType.DMA,
            pltpu.SemaphoreType.DMA,
        ],
        compiler_params=pltpu.CompilerParams(use_tc_tiling_on_sc=False),
    )
    return f(q, srcp, dstp, kp, zeros)


def _final_body(p, fv, lam, gp, pn_o):
    g = gp[0, 0] + gp[1, 0] - gp[0, 1] - gp[1, 1]
    dp = -(fv[...] + g) - lam[...] * p[...]
    pn_o[...] = p[...] + TAU * dp


def _final_call(p, fv, lam, gpart):
    BN = 2000
    grid = N // BN
    return pl.pallas_call(
        _final_body,
        grid=(grid,),
        in_specs=[
            pl.BlockSpec((BN, QD), lambda i: (i, 0)),
            pl.BlockSpec((BN, QD), lambda i: (i, 0)),
            pl.BlockSpec((BN, 1), lambda i: (i, 0)),
            pl.BlockSpec((NC, 2, BN, QD), lambda i: (0, 0, i, 0)),
        ],
        out_specs=pl.BlockSpec((BN, QD), lambda i: (i, 0)),
        out_shape=jax.ShapeDtypeStruct((N, QD), jnp.float32),
    )(p, fv, lam, gpart)


@jax.jit
def kernel(hv_ftr, he_ftr, massive, p_ftr, q_ftr, edge_index,
           W_dis, b_dis, W_pe, b_pe, W_pv, b_pv):
    fv, lam, q_new = _node_call(hv_ftr, p_ftr, q_ftr, massive,
                                W_pv, b_pv, W_dis, b_dis)
    k = _edge_call(he_ftr, W_pe, b_pe)

    pad = EP - E
    srcp = jnp.pad(edge_index[0], (0, pad)).reshape(EP_ROWS, CH)
    dstp = jnp.pad(edge_index[1], (0, pad)).reshape(EP_ROWS, CH)
    zeros = jnp.zeros((N_PAD, QD), jnp.float32)

    gpart = _sc_call(q_ftr, srcp, dstp, k, zeros)
    p_new = _final_call(p_ftr, fv, lam, gpart)
    return (p_new, q_new)
